# baseline (speedup 1.0000x reference)
import functools

import jax
import jax.numpy as jnp
from jax import lax
from jax.experimental import pallas as pl
from jax.experimental.pallas import tpu as pltpu
from jax.experimental.pallas import tpu_sc as plsc

N = 1000000
B = 4096
W = 2 * B - 2
RHO = 0.99
ALPHA = 0.5
NSQ = float(N) ** 2
UPD_SCALE = (1.0 - RHO) * NSQ

ROWS_PER_BLOCK = 128
NUM_BLOCKS = B // ROWS_PER_BLOCK

NUM_TILES = 32
CHUNK = 31256
LAST_CHUNK = N - 31 * CHUNK
VREGS = B // 16


def _tc_body(q1_ref, q2_ref, a1_ref, a2_ref, xim_ref, sa_ref, sr_ref):
    i = pl.program_id(0)
    rs1 = jnp.sum(q1_ref[...], axis=1)
    rs2 = jnp.sum(q2_ref[...], axis=1)
    rsum = rs1 + rs2
    xim_ref[...] = (0.5 * ALPHA) * (a1_ref[...] + a2_ref[...]) \
        + (0.5 * (1.0 - ALPHA) / W) * rsum

    @pl.when(i == 0)
    def _():
        sa_ref[...] = jnp.zeros_like(sa_ref)
        sr_ref[...] = jnp.zeros_like(sr_ref)

    sa_ref[...] = sa_ref[...] + (jnp.sum(a1_ref[...]) + jnp.sum(a2_ref[...]))
    sr_ref[...] = sr_ref[...] + jnp.sum(rsum)


_tc_call = pl.pallas_call(
    _tc_body,
    grid=(NUM_BLOCKS,),
    in_specs=[
        pl.BlockSpec((ROWS_PER_BLOCK, W), lambda i: (i, 0)),
        pl.BlockSpec((ROWS_PER_BLOCK, W), lambda i: (i, 0)),
        pl.BlockSpec((ROWS_PER_BLOCK,), lambda i: (i,)),
        pl.BlockSpec((ROWS_PER_BLOCK,), lambda i: (i,)),
    ],
    out_specs=[
        pl.BlockSpec((ROWS_PER_BLOCK,), lambda i: (i,)),
        pl.BlockSpec((1, 1), lambda i: (0, 0)),
        pl.BlockSpec((1, 1), lambda i: (0, 0)),
    ],
    out_shape=[
        jax.ShapeDtypeStruct((B,), jnp.float32),
        jax.ShapeDtypeStruct((1, 1), jnp.float32),
        jax.ShapeDtypeStruct((1, 1), jnp.float32),
    ],
    compiler_params=pltpu.CompilerParams(
        dimension_semantics=("arbitrary",),
    ),
)


@functools.cache
def _make_sc_scatter():
    return functools.partial(
        pl.kernel,
        mesh=plsc.VectorSubcoreMesh(core_axis_name="c", subcore_axis_name="s"),
        out_type=jax.ShapeDtypeStruct((N,), jnp.float32),
        scratch_types=[
            pltpu.VMEM((CHUNK,), jnp.float32),
            pltpu.VMEM((B,), jnp.int32),
            pltpu.VMEM((B,), jnp.float32),
            pltpu.VMEM((B,), jnp.float32),
            pltpu.VMEM((32,), jnp.int32),
            pltpu.SemaphoreType.DMA,
            pltpu.SemaphoreType.DMA,
            pltpu.SemaphoreType.DMA,
        ],
        compiler_params=pltpu.CompilerParams(needs_layout_passes=False),
    )(_sc_scatter_body)


def _sc_scatter_body(s_inv_hbm, idx_hbm, xim_hbm, out_hbm,
                     chunk_v, idx_v, xim_v, vals_v, nbr_v,
                     sem_i, sem_x, sem_c):
    nc = 2
    wid = lax.axis_index("s") * nc + lax.axis_index("c")
    base = pl.multiple_of(wid * CHUNK, 8)
    is_last = wid == NUM_TILES - 1
    hi = jnp.where(is_last, N, base + CHUNK)

    cp_i = pltpu.async_copy(idx_hbm, idx_v, sem_i)
    cp_x = pltpu.async_copy(xim_hbm, xim_v, sem_x)

    @pl.when(jnp.logical_not(is_last))
    def _():
        pltpu.async_copy(s_inv_hbm.at[pl.ds(base, CHUNK)], chunk_v,
                         sem_c).wait()

    @pl.when(is_last)
    def _():
        pltpu.async_copy(s_inv_hbm.at[pl.ds(31 * CHUNK, LAST_CHUNK)],
                         chunk_v.at[pl.ds(0, LAST_CHUNK)], sem_c).wait()

    cp_i.wait()
    cp_x.wait()

    lane = lax.iota(jnp.int32, 16)

    def pass1(j, _):
        idxv = idx_v[pl.ds(j * 16, 16)]
        inr = jnp.logical_and(idxv >= base, idxv < hi)

        @pl.when(jnp.any(inr))
        def _():
            local = jnp.clip(idxv - base, 0, CHUNK - 1)
            sold = plsc.load_gather(chunk_v, [local], mask=inr)
            vals_v[pl.ds(j * 16, 16)] = RHO * sold \
                + UPD_SCALE * xim_v[pl.ds(j * 16, 16)]

        return 0

    lax.fori_loop(0, VREGS, pass1, 0)

    sent = jnp.int32(1 << 29)
    nbr_v[pl.ds(16, 16)] = jnp.full((16,), -1, jnp.int32)

    def pass2(j, _):
        idxv = idx_v[pl.ds(j * 16, 16)]
        inr = jnp.logical_and(idxv >= base, idxv < hi)

        @pl.when(jnp.any(inr))
        def _():
            local = jnp.clip(idxv - base, 0, CHUNK - 1)
            key = jnp.where(inr, local * 16 + lane, sent + lane)
            sk, sv = plsc.sort_key_val(key, vals_v[pl.ds(j * 16, 16)])
            tgt = lax.shift_right_logical(sk, 4)
            nbr_v[pl.ds(0, 16)] = sk
            nxt = nbr_v[pl.ds(1, 16)]
            keep = tgt != lax.shift_right_logical(nxt, 4)
            mask = jnp.logical_and(keep, sk < sent)
            tgt = jnp.minimum(tgt, CHUNK - 1)
            plsc.store_scatter(chunk_v, [tgt], sv, mask=mask)

        return 0

    lax.fori_loop(0, VREGS, pass2, 0)

    @pl.when(jnp.logical_not(is_last))
    def _():
        pltpu.sync_copy(chunk_v, out_hbm.at[pl.ds(base, CHUNK)])

    @pl.when(is_last)
    def _():
        pltpu.sync_copy(chunk_v.at[pl.ds(0, LAST_CHUNK)],
                        out_hbm.at[pl.ds(31 * CHUNK, LAST_CHUNK)])


def kernel(q_attr_1, q_attr_2, q_rep_1, q_rep_2, feats_idx, s_inv,
           E_attr, E_rep):
    xim, sa, sr = _tc_call(q_rep_1, q_rep_2, q_attr_1, q_attr_2)
    s_inv_new = _make_sc_scatter()(s_inv, feats_idx, xim)
    w = NSQ / (NSQ + 2.0 * B * 100000.0)
    E_attr_new = (1.0 - w) * E_attr + (w / (2.0 * B)) * sa.reshape(1)
    E_rep_new = (1.0 - w) * E_rep + (w / (2.0 * B * W)) * sr.reshape(1)
    return (s_inv_new, E_attr_new, E_rep_new)


# speedup vs baseline: 1.0926x; 1.0926x over previous
import functools

import jax
import jax.numpy as jnp
from jax import lax
from jax.experimental import pallas as pl
from jax.experimental.pallas import tpu as pltpu
from jax.experimental.pallas import tpu_sc as plsc

N = 1000000
B = 4096
W = 2 * B - 2
RHO = 0.99
ALPHA = 0.5
NSQ = float(N) ** 2
UPD_SCALE = (1.0 - RHO) * NSQ

ROWS_PER_BLOCK = 128
NUM_BLOCKS = B // ROWS_PER_BLOCK

NUM_TILES = 32
CHUNK = 31256
LAST_CHUNK = N - 31 * CHUNK
VREGS = B // 16


def _tc_body(q1_ref, q2_ref, a1_ref, a2_ref, xim_ref, sa_ref, sr_ref):
    i = pl.program_id(0)
    rs1 = jnp.sum(q1_ref[...], axis=1)
    rs2 = jnp.sum(q2_ref[...], axis=1)
    rsum = rs1 + rs2
    xim_ref[...] = (0.5 * ALPHA) * (a1_ref[...] + a2_ref[...]) \
        + (0.5 * (1.0 - ALPHA) / W) * rsum

    @pl.when(i == 0)
    def _():
        sa_ref[...] = jnp.zeros_like(sa_ref)
        sr_ref[...] = jnp.zeros_like(sr_ref)

    sa_ref[...] = sa_ref[...] + (jnp.sum(a1_ref[...]) + jnp.sum(a2_ref[...]))
    sr_ref[...] = sr_ref[...] + jnp.sum(rsum)


_tc_call = pl.pallas_call(
    _tc_body,
    grid=(NUM_BLOCKS,),
    in_specs=[
        pl.BlockSpec((ROWS_PER_BLOCK, W), lambda i: (i, 0)),
        pl.BlockSpec((ROWS_PER_BLOCK, W), lambda i: (i, 0)),
        pl.BlockSpec((ROWS_PER_BLOCK,), lambda i: (i,)),
        pl.BlockSpec((ROWS_PER_BLOCK,), lambda i: (i,)),
    ],
    out_specs=[
        pl.BlockSpec((ROWS_PER_BLOCK,), lambda i: (i,)),
        pl.BlockSpec((1, 1), lambda i: (0, 0)),
        pl.BlockSpec((1, 1), lambda i: (0, 0)),
    ],
    out_shape=[
        jax.ShapeDtypeStruct((B,), jnp.float32),
        jax.ShapeDtypeStruct((1, 1), jnp.float32),
        jax.ShapeDtypeStruct((1, 1), jnp.float32),
    ],
    compiler_params=pltpu.CompilerParams(
        dimension_semantics=("arbitrary",),
    ),
)


@functools.cache
def _make_sc_scatter():
    return functools.partial(
        pl.kernel,
        mesh=plsc.VectorSubcoreMesh(core_axis_name="c", subcore_axis_name="s"),
        out_type=jax.ShapeDtypeStruct((N,), jnp.float32),
        scratch_types=[
            pltpu.VMEM((CHUNK,), jnp.float32),
            pltpu.VMEM((B,), jnp.int32),
            pltpu.VMEM((B,), jnp.float32),
            pltpu.VMEM((B,), jnp.float32),
            pltpu.VMEM((32,), jnp.int32),
            pltpu.SemaphoreType.DMA,
            pltpu.SemaphoreType.DMA,
            pltpu.SemaphoreType.DMA,
        ],
        compiler_params=pltpu.CompilerParams(needs_layout_passes=False),
    )(_sc_scatter_body)


def _sc_scatter_body(s_inv_hbm, idx_hbm, xim_hbm, out_hbm,
                     chunk_v, idx_v, xim_v, vals_v, nbr_v,
                     sem_i, sem_x, sem_c):
    nc = 2
    wid = lax.axis_index("s") * nc + lax.axis_index("c")
    base = pl.multiple_of(wid * CHUNK, 8)
    is_last = wid == NUM_TILES - 1
    hi = jnp.where(is_last, N, base + CHUNK)

    cp_i = pltpu.async_copy(idx_hbm, idx_v, sem_i)
    cp_x = pltpu.async_copy(xim_hbm, xim_v, sem_x)

    @pl.when(jnp.logical_not(is_last))
    def _():
        pltpu.async_copy(s_inv_hbm.at[pl.ds(base, CHUNK)], chunk_v,
                         sem_c).wait()

    @pl.when(is_last)
    def _():
        pltpu.async_copy(s_inv_hbm.at[pl.ds(31 * CHUNK, LAST_CHUNK)],
                         chunk_v.at[pl.ds(0, LAST_CHUNK)], sem_c).wait()

    cp_i.wait()
    cp_x.wait()

    lane = lax.iota(jnp.int32, 16)

    def pass1(j, _):
        idxv = idx_v[pl.ds(j * 16, 16)]
        inr = jnp.logical_and(idxv >= base, idxv < hi)
        local = jnp.clip(idxv - base, 0, CHUNK - 1)
        sold = plsc.load_gather(chunk_v, [local], mask=inr)
        vals_v[pl.ds(j * 16, 16)] = RHO * sold \
            + UPD_SCALE * xim_v[pl.ds(j * 16, 16)]
        return 0

    lax.fori_loop(0, VREGS, pass1, 0)

    sent = jnp.int32(1 << 29)
    nbr_v[pl.ds(16, 16)] = jnp.full((16,), -1, jnp.int32)

    def pass2(j, _):
        idxv = idx_v[pl.ds(j * 16, 16)]
        inr = jnp.logical_and(idxv >= base, idxv < hi)
        local = jnp.clip(idxv - base, 0, CHUNK - 1)
        key = jnp.where(inr, local * 16 + lane, sent + lane)
        sk, sv = plsc.sort_key_val(key, vals_v[pl.ds(j * 16, 16)])
        tgt = lax.shift_right_logical(sk, 4)
        nbr_v[pl.ds(0, 16)] = sk
        nxt = nbr_v[pl.ds(1, 16)]
        keep = tgt != lax.shift_right_logical(nxt, 4)
        mask = jnp.logical_and(keep, sk < sent)
        tgt = jnp.minimum(tgt, CHUNK - 1)
        plsc.store_scatter(chunk_v, [tgt], sv, mask=mask)
        return 0

    lax.fori_loop(0, VREGS, pass2, 0)

    @pl.when(jnp.logical_not(is_last))
    def _():
        pltpu.sync_copy(chunk_v, out_hbm.at[pl.ds(base, CHUNK)])

    @pl.when(is_last)
    def _():
        pltpu.sync_copy(chunk_v.at[pl.ds(0, LAST_CHUNK)],
                        out_hbm.at[pl.ds(31 * CHUNK, LAST_CHUNK)])


def kernel(q_attr_1, q_attr_2, q_rep_1, q_rep_2, feats_idx, s_inv,
           E_attr, E_rep):
    xim, sa, sr = _tc_call(q_rep_1, q_rep_2, q_attr_1, q_attr_2)
    s_inv_new = _make_sc_scatter()(s_inv, feats_idx, xim)
    w = NSQ / (NSQ + 2.0 * B * 100000.0)
    E_attr_new = (1.0 - w) * E_attr + (w / (2.0 * B)) * sa.reshape(1)
    E_rep_new = (1.0 - w) * E_rep + (w / (2.0 * B * W)) * sr.reshape(1)
    return (s_inv_new, E_attr_new, E_rep_new)
